# baseline (speedup 1.0000x reference)
import functools

import jax
import jax.numpy as jnp
from jax import lax
from jax.experimental import pallas as pl
from jax.experimental.pallas import tpu as pltpu
from jax.experimental.pallas import tpu_sc as plsc

MAXLEN = 200
EMBED = 64
CHUNK = 400
GATHER = 80
N_GATHER = CHUNK // GATHER


def _make_sc_embed(n_rows):
    info = plsc.get_sparse_core_info()
    nw = info.num_cores * info.num_subcores
    per_w = n_rows // nw
    n_chunks = per_w // CHUNK
    mesh = plsc.VectorSubcoreMesh(core_axis_name="c", subcore_axis_name="s")

    @functools.partial(
        pl.kernel,
        mesh=mesh,
        compiler_params=pltpu.CompilerParams(use_tc_tiling_on_sc=False),
        out_type=jax.ShapeDtypeStruct((n_rows, EMBED), jnp.float32),
        scratch_types=[
            pltpu.VMEM((CHUNK,), jnp.int32),
            pltpu.VMEM((CHUNK, EMBED), jnp.float32),
            pltpu.VMEM((MAXLEN, EMBED), jnp.float32),
            pltpu.SemaphoreType.DMA,
        ],
    )
    def sc_embed(idx_hbm, tab_hbm, pos_hbm, out_hbm, idx_v, rows_v, pos_v, gsem):
        wid = lax.axis_index("s") * info.num_cores + lax.axis_index("c")
        wbase = wid * per_w
        pltpu.sync_copy(pos_hbm, pos_v)

        def chunk_body(g, carry):
            base = wbase + g * CHUNK
            pltpu.sync_copy(idx_hbm.at[pl.ds(base, CHUNK)], idx_v)
            cps = [
                pltpu.async_copy(
                    tab_hbm.at[idx_v.at[pl.ds(j * GATHER, GATHER)]],
                    rows_v.at[pl.ds(j * GATHER, GATHER)],
                    gsem,
                )
                for j in range(N_GATHER)
            ]
            for cp in cps:
                cp.wait()

            def add_body(r, c2):
                for c in range(0, EMBED, 16):
                    pv = pos_v[r, pl.ds(c, 16)]
                    rows_v[r, pl.ds(c, 16)] = rows_v[r, pl.ds(c, 16)] + pv
                    rows_v[r + MAXLEN, pl.ds(c, 16)] = (
                        rows_v[r + MAXLEN, pl.ds(c, 16)] + pv
                    )
                return c2

            lax.fori_loop(0, MAXLEN, add_body, 0)
            pltpu.sync_copy(rows_v, out_hbm.at[pl.ds(base, CHUNK)])
            return carry

        lax.fori_loop(0, n_chunks, chunk_body, 0)

    return sc_embed


def kernel(out, token_table, pos_table):
    batch, maxlen = out.shape
    idx = out.reshape(-1).astype(jnp.int32)
    flat = _make_sc_embed(batch * maxlen)(idx, token_table, pos_table)
    return flat.reshape(batch, maxlen, EMBED)


# speedup vs baseline: 1.1219x; 1.1219x over previous
import functools

import jax
import jax.numpy as jnp
from jax import lax
from jax.experimental import pallas as pl
from jax.experimental.pallas import tpu as pltpu
from jax.experimental.pallas import tpu_sc as plsc

MAXLEN = 200
EMBED = 64
CHUNK = 400
GATHER = 80
N_GATHER = CHUNK // GATHER


def _make_sc_embed(n_rows):
    info = plsc.get_sparse_core_info()
    nw = info.num_cores * info.num_subcores
    per_w = n_rows // nw
    n_chunks = per_w // CHUNK
    mesh = plsc.VectorSubcoreMesh(core_axis_name="c", subcore_axis_name="s")

    @functools.partial(
        pl.kernel,
        mesh=mesh,
        compiler_params=pltpu.CompilerParams(use_tc_tiling_on_sc=False),
        out_type=jax.ShapeDtypeStruct((n_rows, EMBED), jnp.float32),
        scratch_types=[
            pltpu.VMEM((per_w,), jnp.int32),
            pltpu.VMEM((CHUNK, EMBED), jnp.float32),
            pltpu.VMEM((CHUNK, EMBED), jnp.float32),
            pltpu.VMEM((MAXLEN, EMBED), jnp.float32),
            pltpu.SemaphoreType.DMA,
            pltpu.SemaphoreType.DMA,
            pltpu.SemaphoreType.DMA,
            pltpu.SemaphoreType.DMA,
        ],
    )
    def sc_embed(idx_hbm, tab_hbm, pos_hbm, out_hbm,
                 idx_v, rows0, rows1, pos_v, g0, g1, o0, o1):
        wid = lax.axis_index("s") * info.num_cores + lax.axis_index("c")
        wbase = wid * per_w
        rows = (rows0, rows1)
        gsem = (g0, g1)
        osem = (o0, o1)
        pltpu.sync_copy(pos_hbm, pos_v)
        pltpu.sync_copy(idx_hbm.at[pl.ds(wbase, per_w)], idx_v)

        def fire_gathers(g, b):
            for j in range(N_GATHER):
                pltpu.async_copy(
                    tab_hbm.at[idx_v.at[pl.ds(g * CHUNK + j * GATHER, GATHER)]],
                    rows[b].at[pl.ds(j * GATHER, GATHER)],
                    gsem[b],
                )

        def wait_gathers(g, b):
            for j in range(N_GATHER):
                pltpu.make_async_copy(
                    tab_hbm.at[idx_v.at[pl.ds(g * CHUNK + j * GATHER, GATHER)]],
                    rows[b].at[pl.ds(j * GATHER, GATHER)],
                    gsem[b],
                ).wait()

        def wait_writeout(g, b):
            pltpu.make_async_copy(
                rows[b], out_hbm.at[pl.ds(wbase + g * CHUNK, CHUNK)], osem[b]
            ).wait()

        def add_pos(b):
            def add_body(r, c2):
                for c in range(0, EMBED, 16):
                    pv = pos_v[r, pl.ds(c, 16)]
                    rows[b][r, pl.ds(c, 16)] = rows[b][r, pl.ds(c, 16)] + pv
                    rows[b][r + MAXLEN, pl.ds(c, 16)] = (
                        rows[b][r + MAXLEN, pl.ds(c, 16)] + pv
                    )
                return c2

            lax.fori_loop(0, MAXLEN, add_body, 0)

        def fire_writeout(g, b):
            pltpu.async_copy(
                rows[b], out_hbm.at[pl.ds(wbase + g * CHUNK, CHUNK)], osem[b]
            )

        def step(g, b, first=False, last=False):
            if not first:
                wait_writeout(g - 1, 1 - b)
            if not last:
                fire_gathers(g + 1, 1 - b)
            wait_gathers(g, b)
            add_pos(b)
            fire_writeout(g, b)

        fire_gathers(0, 0)
        step(0, 0, first=True)

        def outer(i0, carry):
            step(2 * i0 + 1, 1)
            step(2 * i0 + 2, 0)
            return carry

        lax.fori_loop(0, n_chunks // 2 - 1, outer, 0)
        step(n_chunks - 1, 1, last=True)
        wait_writeout(n_chunks - 1, 1)

    return sc_embed


def kernel(out, token_table, pos_table):
    batch, maxlen = out.shape
    idx = out.reshape(-1).astype(jnp.int32)
    flat = _make_sc_embed(batch * maxlen)(idx, token_table, pos_table)
    return flat.reshape(batch, maxlen, EMBED)


# speedup vs baseline: 1.3749x; 1.2255x over previous
import functools

import jax
import jax.numpy as jnp
from jax import lax
from jax.experimental import pallas as pl
from jax.experimental.pallas import tpu as pltpu
from jax.experimental.pallas import tpu_sc as plsc

MAXLEN = 200
EMBED = 64
LANES = 128
CHUNK = 200
SUBG = (128, 72)


def _make_sc_embed(n_rows):
    info = plsc.get_sparse_core_info()
    nw = info.num_cores * info.num_subcores
    per_w = n_rows // nw
    n_chunks = per_w // CHUNK
    mesh = plsc.VectorSubcoreMesh(core_axis_name="c", subcore_axis_name="s")

    @functools.partial(
        pl.kernel,
        mesh=mesh,
        compiler_params=pltpu.CompilerParams(use_tc_tiling_on_sc=True),
        out_type=jax.ShapeDtypeStruct((n_rows, LANES), jnp.float32),
        scratch_types=[
            pltpu.VMEM((per_w,), jnp.int32),
            pltpu.VMEM((CHUNK, LANES), jnp.float32),
            pltpu.VMEM((CHUNK, LANES), jnp.float32),
            pltpu.VMEM((MAXLEN, EMBED), jnp.float32),
            pltpu.SemaphoreType.DMA,
            pltpu.SemaphoreType.DMA,
            pltpu.SemaphoreType.DMA,
            pltpu.SemaphoreType.DMA,
        ],
    )
    def sc_embed(idx_hbm, tab_hbm, pos_hbm, out_hbm,
                 idx_v, rows0, rows1, pos_v, g0, g1, o0, o1):
        wid = lax.axis_index("s") * info.num_cores + lax.axis_index("c")
        wbase = wid * per_w
        rows = (rows0, rows1)
        gsem = (g0, g1)
        osem = (o0, o1)
        pltpu.sync_copy(pos_hbm, pos_v)
        pltpu.sync_copy(idx_hbm.at[pl.ds(wbase, per_w)], idx_v)

        def fire_gathers(g, b):
            off = 0
            for n in SUBG:
                pltpu.async_copy(
                    tab_hbm.at[idx_v.at[pl.ds(g * CHUNK + off, n)]],
                    rows[b].at[pl.ds(off, n)],
                    gsem[b],
                )
                off += n

        def wait_gathers(g, b):
            off = 0
            for n in SUBG:
                pltpu.make_async_copy(
                    tab_hbm.at[idx_v.at[pl.ds(g * CHUNK + off, n)]],
                    rows[b].at[pl.ds(off, n)],
                    gsem[b],
                ).wait()
                off += n

        def wait_writeout(g, b):
            pltpu.make_async_copy(
                rows[b],
                out_hbm.at[pl.ds(wbase + g * CHUNK, CHUNK)],
                osem[b],
            ).wait()

        def add_pos(b):
            def add_body(t, c2):
                for c in range(0, EMBED, 16):
                    rows[b][t, pl.ds(c, 16)] = (
                        rows[b][t, pl.ds(c, 16)] + pos_v[t, pl.ds(c, 16)]
                    )
                return c2

            lax.fori_loop(0, CHUNK, add_body, 0)

        def fire_writeout(g, b):
            pltpu.async_copy(
                rows[b],
                out_hbm.at[pl.ds(wbase + g * CHUNK, CHUNK)],
                osem[b],
            )

        def step(g, b, first=False, last=False):
            if not first:
                wait_writeout(g - 1, 1 - b)
            if not last:
                fire_gathers(g + 1, 1 - b)
            wait_gathers(g, b)
            add_pos(b)
            fire_writeout(g, b)

        fire_gathers(0, 0)
        step(0, 0, first=True)

        def outer(i0, carry):
            step(2 * i0 + 1, 1)
            step(2 * i0 + 2, 0)
            return carry

        lax.fori_loop(0, n_chunks // 2 - 1, outer, 0)
        step(n_chunks - 1, 1, last=True)
        wait_writeout(n_chunks - 1, 1)

    return sc_embed


def kernel(out, token_table, pos_table):
    batch, maxlen = out.shape
    idx = out.reshape(-1).astype(jnp.int32)
    tab128 = jnp.pad(token_table, ((0, 0), (0, LANES - EMBED)))
    flat = _make_sc_embed(batch * maxlen)(idx, tab128, pos_table)
    return flat[:, :EMBED].reshape(batch, maxlen, EMBED)


# speedup vs baseline: 1.3781x; 1.0024x over previous
import functools

import jax
import jax.numpy as jnp
from jax import lax
from jax.experimental import pallas as pl
from jax.experimental.pallas import tpu as pltpu
from jax.experimental.pallas import tpu_sc as plsc

MAXLEN = 200
EMBED = 64
LANES = 128
CHUNK = 200
SUBG = (128, 72)


def _make_sc_embed(n_rows):
    info = plsc.get_sparse_core_info()
    nw = info.num_cores * info.num_subcores
    per_w = n_rows // nw
    n_chunks = per_w // CHUNK
    mesh = plsc.VectorSubcoreMesh(core_axis_name="c", subcore_axis_name="s")

    @functools.partial(
        pl.kernel,
        mesh=mesh,
        compiler_params=pltpu.CompilerParams(use_tc_tiling_on_sc=True),
        out_type=jax.ShapeDtypeStruct((n_rows, LANES), jnp.float32),
        scratch_types=[
            pltpu.VMEM((per_w,), jnp.int32),
            pltpu.VMEM((CHUNK, LANES), jnp.float32),
            pltpu.VMEM((CHUNK, LANES), jnp.float32),
            pltpu.VMEM((MAXLEN, EMBED), jnp.float32),
            pltpu.SemaphoreType.DMA,
            pltpu.SemaphoreType.DMA,
            pltpu.SemaphoreType.DMA,
            pltpu.SemaphoreType.DMA,
        ],
    )
    def sc_embed(idx_hbm, tab_hbm, pos_hbm, out_hbm,
                 idx_v, rows0, rows1, pos_v, g0, g1, o0, o1):
        wid = lax.axis_index("s") * info.num_cores + lax.axis_index("c")
        wbase = wid * per_w
        rows = (rows0, rows1)
        gsem = (g0, g1)
        osem = (o0, o1)
        pltpu.sync_copy(pos_hbm, pos_v)
        pltpu.sync_copy(idx_hbm.at[pl.ds(wbase, per_w)], idx_v)

        def fire_gathers(g, b):
            off = 0
            for n in SUBG:
                pltpu.async_copy(
                    tab_hbm.at[idx_v.at[pl.ds(g * CHUNK + off, n)]],
                    rows[b].at[pl.ds(off, n)],
                    gsem[b],
                )
                off += n

        def wait_gathers(g, b):
            off = 0
            for n in SUBG:
                pltpu.make_async_copy(
                    tab_hbm.at[idx_v.at[pl.ds(g * CHUNK + off, n)]],
                    rows[b].at[pl.ds(off, n)],
                    gsem[b],
                ).wait()
                off += n

        def wait_writeout(g, b):
            pltpu.make_async_copy(
                rows[b],
                out_hbm.at[pl.ds(wbase + g * CHUNK, CHUNK)],
                osem[b],
            ).wait()

        def add_pos(b):
            @plsc.parallel_loop(0, CHUNK, step=1, unroll=8)
            def _(t):
                for c in range(0, EMBED, 16):
                    rows[b][t, pl.ds(c, 16)] = (
                        rows[b][t, pl.ds(c, 16)] + pos_v[t, pl.ds(c, 16)]
                    )

        def fire_writeout(g, b):
            pltpu.async_copy(
                rows[b],
                out_hbm.at[pl.ds(wbase + g * CHUNK, CHUNK)],
                osem[b],
            )

        def step(g, b, first=False, last=False):
            if not first:
                wait_writeout(g - 1, 1 - b)
            if not last:
                fire_gathers(g + 1, 1 - b)
            wait_gathers(g, b)
            add_pos(b)
            fire_writeout(g, b)

        fire_gathers(0, 0)
        step(0, 0, first=True)

        def outer(i0, carry):
            step(2 * i0 + 1, 1)
            step(2 * i0 + 2, 0)
            return carry

        lax.fori_loop(0, n_chunks // 2 - 1, outer, 0)
        step(n_chunks - 1, 1, last=True)
        wait_writeout(n_chunks - 1, 1)

    return sc_embed


def kernel(out, token_table, pos_table):
    batch, maxlen = out.shape
    idx = out.reshape(-1).astype(jnp.int32)
    tab128 = jnp.pad(token_table, ((0, 0), (0, LANES - EMBED)))
    flat = _make_sc_embed(batch * maxlen)(idx, tab128, pos_table)
    return flat[:, :EMBED].reshape(batch, maxlen, EMBED)
